# baseline (speedup 1.0000x reference)
import functools

import jax
import jax.numpy as jnp
from jax import lax
from jax.experimental import pallas as pl
from jax.experimental.pallas import tpu as pltpu
from jax.experimental.pallas import tpu_sc as plsc

N = 10000
E = 320000
D = 128
R = 32
NB = 4

BLK = 400
NGRID = N // BLK

NWORKER = 32
CH = 112
NCHUNKS = (E + CH - 1) // CH
JTRIPS = (NCHUNKS + NWORKER - 1) // NWORKER
ZCH = 400
NZ = N // ZCH


def _y_body(h_ref, basis_ref, comp_ref, y_ref, w_scr):
    @pl.when(pl.program_id(0) == 0)
    def _():
        for r in range(R):
            w = comp_ref[r, 0] * basis_ref[0]
            for b in range(1, NB):
                w = w + comp_ref[r, b] * basis_ref[b]
            w_scr[r] = w

    h = h_ref[...]
    for r in range(R):
        y_ref[:, r, :] = jnp.dot(h, w_scr[r], preferred_element_type=jnp.float32)


def _project(h, basis, comp):
    return pl.pallas_call(
        _y_body,
        grid=(NGRID,),
        in_specs=[
            pl.BlockSpec((BLK, D), lambda i: (i, 0)),
            pl.BlockSpec((NB, D, D), lambda i: (0, 0, 0)),
            pl.BlockSpec(memory_space=pltpu.SMEM),
        ],
        out_specs=pl.BlockSpec((BLK, R, D), lambda i: (i, 0, 0)),
        out_shape=jax.ShapeDtypeStruct((N, R, D), jnp.float32),
        scratch_shapes=[pltpu.VMEM((R, D, D), jnp.float32)],
    )(h, basis, comp)


def _cy_body(h_ref, acc_ref, wself_ref, basis_ref, comp_ref, hn_ref, y_ref, w_scr):
    @pl.when(pl.program_id(0) == 0)
    def _():
        for r in range(R):
            w = comp_ref[r, 0] * basis_ref[0]
            for b in range(1, NB):
                w = w + comp_ref[r, b] * basis_ref[b]
            w_scr[r] = w

    hn = acc_ref[0] + acc_ref[1] + jnp.dot(
        h_ref[...], wself_ref[...], preferred_element_type=jnp.float32)
    hn = jnp.maximum(hn, 0.0)
    hn_ref[...] = hn
    for r in range(R):
        y_ref[:, r, :] = jnp.dot(hn, w_scr[r], preferred_element_type=jnp.float32)


def _combine_project(h, acc, wself, basis, comp):
    return pl.pallas_call(
        _cy_body,
        grid=(NGRID,),
        in_specs=[
            pl.BlockSpec((BLK, D), lambda i: (i, 0)),
            pl.BlockSpec((2, BLK, D), lambda i: (0, i, 0)),
            pl.BlockSpec((D, D), lambda i: (0, 0)),
            pl.BlockSpec((NB, D, D), lambda i: (0, 0, 0)),
            pl.BlockSpec(memory_space=pltpu.SMEM),
        ],
        out_specs=[
            pl.BlockSpec((BLK, D), lambda i: (i, 0)),
            pl.BlockSpec((BLK, R, D), lambda i: (i, 0, 0)),
        ],
        out_shape=[
            jax.ShapeDtypeStruct((N, D), jnp.float32),
            jax.ShapeDtypeStruct((N, R, D), jnp.float32),
        ],
        scratch_shapes=[pltpu.VMEM((R, D, D), jnp.float32)],
    )(h, acc, wself, basis, comp)


def _fin_body(h_ref, acc_ref, wself_ref, h3_ref, stats_ref, s1_scr, s2_scr):
    h3 = acc_ref[0] + acc_ref[1] + jnp.dot(
        h_ref[...], wself_ref[...], preferred_element_type=jnp.float32)
    h3 = jnp.maximum(h3, 0.0)
    h3_ref[...] = h3

    @pl.when(pl.program_id(0) == 0)
    def _():
        s1_scr[...] = jnp.zeros((8, D), jnp.float32)
        s2_scr[...] = jnp.zeros((8, D), jnp.float32)

    s1_scr[...] += jnp.sum(h3.reshape(BLK // 8, 8, D), axis=0)
    s2_scr[...] += jnp.sum((h3 * h3).reshape(BLK // 8, 8, D), axis=0)
    stats_ref[0] = s1_scr[...]
    stats_ref[1] = s2_scr[...]


def _finalize(h, acc, wself):
    return pl.pallas_call(
        _fin_body,
        grid=(NGRID,),
        in_specs=[
            pl.BlockSpec((BLK, D), lambda i: (i, 0)),
            pl.BlockSpec((2, BLK, D), lambda i: (0, i, 0)),
            pl.BlockSpec((D, D), lambda i: (0, 0)),
        ],
        out_specs=[
            pl.BlockSpec((BLK, D), lambda i: (i, 0)),
            pl.BlockSpec((2, 8, D), lambda i: (0, 0, 0)),
        ],
        out_shape=[
            jax.ShapeDtypeStruct((N, D), jnp.float32),
            jax.ShapeDtypeStruct((2, 8, D), jnp.float32),
        ],
        scratch_shapes=[
            pltpu.VMEM((8, D), jnp.float32),
            pltpu.VMEM((8, D), jnp.float32),
        ],
    )(h, acc, wself)


def _bn_body(h3_ref, h2_ref, stats_ref, gamma_ref, beta_ref, out_ref):
    inv_n = 1.0 / N
    mean = jnp.sum(stats_ref[0], axis=0, keepdims=True) * inv_n
    ex2 = jnp.sum(stats_ref[1], axis=0, keepdims=True) * inv_n
    var = ex2 - mean * mean
    inv = lax.rsqrt(var + 1e-5)
    out_ref[...] = h2_ref[...] + (h3_ref[...] - mean) * inv * gamma_ref[...] \
        + beta_ref[...]


def _batchnorm_residual(h3, h2, stats, gamma, beta):
    return pl.pallas_call(
        _bn_body,
        grid=(NGRID,),
        in_specs=[
            pl.BlockSpec((BLK, D), lambda i: (i, 0)),
            pl.BlockSpec((BLK, D), lambda i: (i, 0)),
            pl.BlockSpec((2, 8, D), lambda i: (0, 0, 0)),
            pl.BlockSpec((1, D), lambda i: (0, 0)),
            pl.BlockSpec((1, D), lambda i: (0, 0)),
        ],
        out_specs=pl.BlockSpec((BLK, D), lambda i: (i, 0)),
        out_shape=jax.ShapeDtypeStruct((N, D), jnp.float32),
    )(h3, h2, stats, gamma, beta)


_SC_MESH = plsc.VectorSubcoreMesh(core_axis_name="c", subcore_axis_name="s")
ROT = 3
_GSPLIT = ((0, 32), (32, 32), (64, 32), (96, 16))


@functools.partial(
    pl.kernel,
    out_type=jax.ShapeDtypeStruct((2, N, D), jnp.float32),
    mesh=_SC_MESH,
    scratch_types=[
        pltpu.VMEM((4, CH), jnp.int32),
        pltpu.VMEM((4, CH), jnp.int32),
        pltpu.VMEM((4, CH), jnp.int32),
        pltpu.VMEM((CH,), jnp.int32),
        pltpu.VMEM((CH,), jnp.int32),
        pltpu.VMEM((CH,), jnp.int32),
        pltpu.VMEM((CH,), jnp.int32),
        pltpu.VMEM((CH,), jnp.int32),
        pltpu.VMEM((CH,), jnp.int32),
        pltpu.VMEM((CH, D), jnp.float32),
        pltpu.VMEM((CH, D), jnp.float32),
        pltpu.VMEM((CH, D), jnp.float32),
        pltpu.VMEM_SHARED((N, D), jnp.float32),
        pltpu.SemaphoreType.DMA,
        pltpu.SemaphoreType.DMA,
        pltpu.SemaphoreType.DMA,
        pltpu.SemaphoreType.DMA,
        pltpu.SemaphoreType.DMA,
        pltpu.SemaphoreType.DMA,
    ],
)
def _sc_edge_kernel(y_hbm, edata_hbm, zeros_hbm, out_hbm,
                    ed0, ed1, ed2, idx0, idx1, idx2, dst0, dst1, dst2,
                    rows0, rows1, rows2, acc_sh,
                    sg0, sg1, sg2, ss0, ss1, ss2):
    cid = lax.axis_index("c")
    sid = lax.axis_index("s")
    wid = cid * 16 + sid
    eds = (ed0, ed1, ed2)
    idxs = (idx0, idx1, idx2)
    dsts = (dst0, dst1, dst2)
    rows = (rows0, rows1, rows2)
    sgs = (sg0, sg1, sg2)
    sss = (ss0, ss1, ss2)

    for j in range((NZ + 15) // 16):
        zc = j * 16 + sid

        @pl.when(zc < NZ)
        def _(zc=zc):
            pltpu.sync_copy(zeros_hbm, acc_sh.at[pl.ds(zc * ZCH, ZCH)])

    plsc.subcore_barrier()

    def _prefetch(c, b):
        pltpu.sync_copy(edata_hbm.at[c], eds[b])

        def idx_body(i, cc):
            sl = pl.ds(i * 16, 16)
            idxs[b][sl] = eds[b][0, sl] * R + eds[b][1, sl]
            dsts[b][sl] = eds[b][2, sl]
            return cc

        lax.fori_loop(0, CH // 16, idx_body, 0, unroll=CH // 16)
        for (o, l) in _GSPLIT:
            pltpu.async_copy(y_hbm.at[idxs[b].at[pl.ds(o, l)]],
                             rows[b].at[pl.ds(o, l)], sgs[b])

    def _scale_and_scatter(b):
        for (o, l) in _GSPLIT:
            pltpu.make_async_copy(y_hbm.at[idxs[b].at[pl.ds(o, l)]],
                                  rows[b].at[pl.ds(o, l)], sgs[b]).wait()

        def scale_body(kk, cc):
            sl16 = pl.ds(kk * 16, 16)
            n16 = eds[b][3, sl16].astype(jnp.float32) * (1.0 / 16777216.0)
            for e in range(16):
                k = kk * 16 + e
                nv = n16[e]
                for j in range(D // 16):
                    sl = pl.ds(j * 16, 16)
                    rows[b][k, sl] = rows[b][k, sl] * nv
            return cc

        lax.fori_loop(0, CH // 16, scale_body, 0)
        pltpu.async_copy(rows[b], acc_sh.at[dsts[b]], sss[b], add=True)

    def _wait_scatter(b):
        pltpu.make_async_copy(rows[b], acc_sh.at[dsts[b]], sss[b]).wait()

    def chunk_body(j, carry):
        c = j * NWORKER + wid

        @pl.when(j == 0)
        def _():
            _prefetch(c, 0)
            _prefetch(c + NWORKER, 1)

        for b in range(ROT):
            is_b = lax.rem(j, ROT) == b
            bp = (b + 2) % ROT

            @pl.when(is_b & (j >= 1) & (c - NWORKER < NCHUNKS))
            def _(bp=bp):
                _wait_scatter(bp)

            @pl.when(is_b & (c + 2 * NWORKER < NCHUNKS))
            def _(c=c, bp=bp):
                _prefetch(c + 2 * NWORKER, bp)

            @pl.when(is_b & (c < NCHUNKS))
            def _(b=b):
                _scale_and_scatter(b)

        return carry

    lax.fori_loop(0, JTRIPS, chunk_body, 0)

    @pl.when((JTRIPS - 1) * NWORKER + wid < NCHUNKS)
    def _():
        _wait_scatter((JTRIPS - 1) % ROT)

    plsc.subcore_barrier()
    for j in range((NZ + 15) // 16):
        zc = j * 16 + sid

        @pl.when(zc < NZ)
        def _(zc=zc):
            pltpu.sync_copy(acc_sh.at[pl.ds(zc * ZCH, ZCH)],
                            out_hbm.at[cid, pl.ds(zc * ZCH, ZCH)])


def kernel(x, edge_index, edge_type, norm, basis0, comp0, wself0, basis1,
           comp1, wself1, basis2, comp2, wself2, gamma, beta):
    src = edge_index[0]
    dst = edge_index[1]
    zeros = jnp.zeros((ZCH, D), jnp.float32)
    normq = (norm * 16777216.0).astype(jnp.int32)
    pad = NCHUNKS * CH - E

    def _padded(a):
        return jnp.concatenate([a, jnp.zeros((pad,), jnp.int32)]).reshape(
            NCHUNKS, CH)

    edata = jnp.stack(
        [_padded(src), _padded(edge_type), _padded(dst), _padded(normq)],
        axis=1)

    y0 = _project(x, basis0, comp0)
    acc0 = _sc_edge_kernel(y0.reshape(N * R, D), edata, zeros)
    h1, y1 = _combine_project(x, acc0, wself0, basis1, comp1)
    acc1 = _sc_edge_kernel(y1.reshape(N * R, D), edata, zeros)
    h2, y2 = _combine_project(h1, acc1, wself1, basis2, comp2)
    acc2 = _sc_edge_kernel(y2.reshape(N * R, D), edata, zeros)
    h3, stats = _finalize(h2, acc2, wself2)
    out = _batchnorm_residual(h3, h2, stats, gamma.reshape(1, D),
                              beta.reshape(1, D))
    return out


# speedup vs baseline: 1.0001x; 1.0001x over previous
import functools

import jax
import jax.numpy as jnp
from jax import lax
from jax.experimental import pallas as pl
from jax.experimental.pallas import tpu as pltpu
from jax.experimental.pallas import tpu_sc as plsc

N = 10000
E = 320000
D = 128
R = 32
NB = 4

BLK = 400
NGRID = N // BLK

NWORKER = 32
CH = 112
NCHUNKS = (E + CH - 1) // CH
JTRIPS = (NCHUNKS + NWORKER - 1) // NWORKER
ZCH = 400
NZ = N // ZCH


def _y_body(h_ref, basis_ref, comp_ref, y_ref, w_scr):
    @pl.when(pl.program_id(0) == 0)
    def _():
        for r in range(R):
            w = comp_ref[r, 0] * basis_ref[0]
            for b in range(1, NB):
                w = w + comp_ref[r, b] * basis_ref[b]
            w_scr[r] = w

    h = h_ref[...]
    for r in range(R):
        y_ref[:, r, :] = jnp.dot(h, w_scr[r], preferred_element_type=jnp.float32)


def _project(h, basis, comp):
    return pl.pallas_call(
        _y_body,
        grid=(NGRID,),
        in_specs=[
            pl.BlockSpec((BLK, D), lambda i: (i, 0)),
            pl.BlockSpec((NB, D, D), lambda i: (0, 0, 0)),
            pl.BlockSpec(memory_space=pltpu.SMEM),
        ],
        out_specs=pl.BlockSpec((BLK, R, D), lambda i: (i, 0, 0)),
        out_shape=jax.ShapeDtypeStruct((N, R, D), jnp.float32),
        scratch_shapes=[pltpu.VMEM((R, D, D), jnp.float32)],
    )(h, basis, comp)


def _cy_body(h_ref, acc_ref, wself_ref, basis_ref, comp_ref, hn_ref, y_ref, w_scr):
    @pl.when(pl.program_id(0) == 0)
    def _():
        for r in range(R):
            w = comp_ref[r, 0] * basis_ref[0]
            for b in range(1, NB):
                w = w + comp_ref[r, b] * basis_ref[b]
            w_scr[r] = w

    hn = acc_ref[0] + acc_ref[1] + jnp.dot(
        h_ref[...], wself_ref[...], preferred_element_type=jnp.float32)
    hn = jnp.maximum(hn, 0.0)
    hn_ref[...] = hn
    for r in range(R):
        y_ref[:, r, :] = jnp.dot(hn, w_scr[r], preferred_element_type=jnp.float32)


def _combine_project(h, acc, wself, basis, comp):
    return pl.pallas_call(
        _cy_body,
        grid=(NGRID,),
        in_specs=[
            pl.BlockSpec((BLK, D), lambda i: (i, 0)),
            pl.BlockSpec((2, BLK, D), lambda i: (0, i, 0)),
            pl.BlockSpec((D, D), lambda i: (0, 0)),
            pl.BlockSpec((NB, D, D), lambda i: (0, 0, 0)),
            pl.BlockSpec(memory_space=pltpu.SMEM),
        ],
        out_specs=[
            pl.BlockSpec((BLK, D), lambda i: (i, 0)),
            pl.BlockSpec((BLK, R, D), lambda i: (i, 0, 0)),
        ],
        out_shape=[
            jax.ShapeDtypeStruct((N, D), jnp.float32),
            jax.ShapeDtypeStruct((N, R, D), jnp.float32),
        ],
        scratch_shapes=[pltpu.VMEM((R, D, D), jnp.float32)],
    )(h, acc, wself, basis, comp)


def _fin_body(h_ref, acc_ref, wself_ref, h3_ref, stats_ref, s1_scr, s2_scr):
    h3 = acc_ref[0] + acc_ref[1] + jnp.dot(
        h_ref[...], wself_ref[...], preferred_element_type=jnp.float32)
    h3 = jnp.maximum(h3, 0.0)
    h3_ref[...] = h3

    @pl.when(pl.program_id(0) == 0)
    def _():
        s1_scr[...] = jnp.zeros((8, D), jnp.float32)
        s2_scr[...] = jnp.zeros((8, D), jnp.float32)

    s1_scr[...] += jnp.sum(h3.reshape(BLK // 8, 8, D), axis=0)
    s2_scr[...] += jnp.sum((h3 * h3).reshape(BLK // 8, 8, D), axis=0)
    stats_ref[0] = s1_scr[...]
    stats_ref[1] = s2_scr[...]


def _finalize(h, acc, wself):
    return pl.pallas_call(
        _fin_body,
        grid=(NGRID,),
        in_specs=[
            pl.BlockSpec((BLK, D), lambda i: (i, 0)),
            pl.BlockSpec((2, BLK, D), lambda i: (0, i, 0)),
            pl.BlockSpec((D, D), lambda i: (0, 0)),
        ],
        out_specs=[
            pl.BlockSpec((BLK, D), lambda i: (i, 0)),
            pl.BlockSpec((2, 8, D), lambda i: (0, 0, 0)),
        ],
        out_shape=[
            jax.ShapeDtypeStruct((N, D), jnp.float32),
            jax.ShapeDtypeStruct((2, 8, D), jnp.float32),
        ],
        scratch_shapes=[
            pltpu.VMEM((8, D), jnp.float32),
            pltpu.VMEM((8, D), jnp.float32),
        ],
    )(h, acc, wself)


def _bn_body(h3_ref, h2_ref, stats_ref, gamma_ref, beta_ref, out_ref):
    inv_n = 1.0 / N
    mean = jnp.sum(stats_ref[0], axis=0, keepdims=True) * inv_n
    ex2 = jnp.sum(stats_ref[1], axis=0, keepdims=True) * inv_n
    var = ex2 - mean * mean
    inv = lax.rsqrt(var + 1e-5)
    out_ref[...] = h2_ref[...] + (h3_ref[...] - mean) * inv * gamma_ref[...] \
        + beta_ref[...]


def _batchnorm_residual(h3, h2, stats, gamma, beta):
    return pl.pallas_call(
        _bn_body,
        grid=(NGRID,),
        in_specs=[
            pl.BlockSpec((BLK, D), lambda i: (i, 0)),
            pl.BlockSpec((BLK, D), lambda i: (i, 0)),
            pl.BlockSpec((2, 8, D), lambda i: (0, 0, 0)),
            pl.BlockSpec((1, D), lambda i: (0, 0)),
            pl.BlockSpec((1, D), lambda i: (0, 0)),
        ],
        out_specs=pl.BlockSpec((BLK, D), lambda i: (i, 0)),
        out_shape=jax.ShapeDtypeStruct((N, D), jnp.float32),
    )(h3, h2, stats, gamma, beta)


_SC_MESH = plsc.VectorSubcoreMesh(core_axis_name="c", subcore_axis_name="s")
ROT = 3
_GSPLIT = ((0, CH),)


@functools.partial(
    pl.kernel,
    out_type=jax.ShapeDtypeStruct((2, N, D), jnp.float32),
    mesh=_SC_MESH,
    scratch_types=[
        pltpu.VMEM((4, CH), jnp.int32),
        pltpu.VMEM((4, CH), jnp.int32),
        pltpu.VMEM((4, CH), jnp.int32),
        pltpu.VMEM((CH,), jnp.int32),
        pltpu.VMEM((CH,), jnp.int32),
        pltpu.VMEM((CH,), jnp.int32),
        pltpu.VMEM((CH,), jnp.int32),
        pltpu.VMEM((CH,), jnp.int32),
        pltpu.VMEM((CH,), jnp.int32),
        pltpu.VMEM((CH, D), jnp.float32),
        pltpu.VMEM((CH, D), jnp.float32),
        pltpu.VMEM((CH, D), jnp.float32),
        pltpu.VMEM_SHARED((N, D), jnp.float32),
        pltpu.SemaphoreType.DMA,
        pltpu.SemaphoreType.DMA,
        pltpu.SemaphoreType.DMA,
        pltpu.SemaphoreType.DMA,
        pltpu.SemaphoreType.DMA,
        pltpu.SemaphoreType.DMA,
    ],
)
def _sc_edge_kernel(y_hbm, edata_hbm, zeros_hbm, out_hbm,
                    ed0, ed1, ed2, idx0, idx1, idx2, dst0, dst1, dst2,
                    rows0, rows1, rows2, acc_sh,
                    sg0, sg1, sg2, ss0, ss1, ss2):
    cid = lax.axis_index("c")
    sid = lax.axis_index("s")
    wid = cid * 16 + sid
    eds = (ed0, ed1, ed2)
    idxs = (idx0, idx1, idx2)
    dsts = (dst0, dst1, dst2)
    rows = (rows0, rows1, rows2)
    sgs = (sg0, sg1, sg2)
    sss = (ss0, ss1, ss2)

    for j in range((NZ + 15) // 16):
        zc = j * 16 + sid

        @pl.when(zc < NZ)
        def _(zc=zc):
            pltpu.sync_copy(zeros_hbm, acc_sh.at[pl.ds(zc * ZCH, ZCH)])

    plsc.subcore_barrier()

    def _prefetch(c, b):
        pltpu.sync_copy(edata_hbm.at[c], eds[b])

        def idx_body(i, cc):
            sl = pl.ds(i * 16, 16)
            idxs[b][sl] = eds[b][0, sl] * R + eds[b][1, sl]
            dsts[b][sl] = eds[b][2, sl]
            return cc

        lax.fori_loop(0, CH // 16, idx_body, 0, unroll=CH // 16)
        for (o, l) in _GSPLIT:
            pltpu.async_copy(y_hbm.at[idxs[b].at[pl.ds(o, l)]],
                             rows[b].at[pl.ds(o, l)], sgs[b])

    def _scale_and_scatter(b):
        for (o, l) in _GSPLIT:
            pltpu.make_async_copy(y_hbm.at[idxs[b].at[pl.ds(o, l)]],
                                  rows[b].at[pl.ds(o, l)], sgs[b]).wait()

        def scale_body(kk, cc):
            sl16 = pl.ds(kk * 16, 16)
            n16 = eds[b][3, sl16].astype(jnp.float32) * (1.0 / 16777216.0)
            for e in range(16):
                k = kk * 16 + e
                nv = n16[e]
                for j in range(D // 16):
                    sl = pl.ds(j * 16, 16)
                    rows[b][k, sl] = rows[b][k, sl] * nv
            return cc

        lax.fori_loop(0, CH // 16, scale_body, 0)
        pltpu.async_copy(rows[b], acc_sh.at[dsts[b]], sss[b], add=True)

    def _wait_scatter(b):
        pltpu.make_async_copy(rows[b], acc_sh.at[dsts[b]], sss[b]).wait()

    def chunk_body(j, carry):
        c = j * NWORKER + wid

        @pl.when(j == 0)
        def _():
            _prefetch(c, 0)
            _prefetch(c + NWORKER, 1)

        for b in range(ROT):
            is_b = lax.rem(j, ROT) == b
            bp = (b + 2) % ROT

            @pl.when(is_b & (j >= 1) & (c - NWORKER < NCHUNKS))
            def _(bp=bp):
                _wait_scatter(bp)

            @pl.when(is_b & (c + 2 * NWORKER < NCHUNKS))
            def _(c=c, bp=bp):
                _prefetch(c + 2 * NWORKER, bp)

            @pl.when(is_b & (c < NCHUNKS))
            def _(b=b):
                _scale_and_scatter(b)

        return carry

    lax.fori_loop(0, JTRIPS, chunk_body, 0)

    @pl.when((JTRIPS - 1) * NWORKER + wid < NCHUNKS)
    def _():
        _wait_scatter((JTRIPS - 1) % ROT)

    plsc.subcore_barrier()
    for j in range((NZ + 15) // 16):
        zc = j * 16 + sid

        @pl.when(zc < NZ)
        def _(zc=zc):
            pltpu.sync_copy(acc_sh.at[pl.ds(zc * ZCH, ZCH)],
                            out_hbm.at[cid, pl.ds(zc * ZCH, ZCH)])


def kernel(x, edge_index, edge_type, norm, basis0, comp0, wself0, basis1,
           comp1, wself1, basis2, comp2, wself2, gamma, beta):
    src = edge_index[0]
    dst = edge_index[1]
    zeros = jnp.zeros((ZCH, D), jnp.float32)
    normq = (norm * 16777216.0).astype(jnp.int32)
    pad = NCHUNKS * CH - E

    def _padded(a):
        return jnp.concatenate([a, jnp.zeros((pad,), jnp.int32)]).reshape(
            NCHUNKS, CH)

    edata = jnp.stack(
        [_padded(src), _padded(edge_type), _padded(dst), _padded(normq)],
        axis=1)

    y0 = _project(x, basis0, comp0)
    acc0 = _sc_edge_kernel(y0.reshape(N * R, D), edata, zeros)
    h1, y1 = _combine_project(x, acc0, wself0, basis1, comp1)
    acc1 = _sc_edge_kernel(y1.reshape(N * R, D), edata, zeros)
    h2, y2 = _combine_project(h1, acc1, wself1, basis2, comp2)
    acc2 = _sc_edge_kernel(y2.reshape(N * R, D), edata, zeros)
    h3, stats = _finalize(h2, acc2, wself2)
    out = _batchnorm_residual(h3, h2, stats, gamma.reshape(1, D),
                              beta.reshape(1, D))
    return out


# speedup vs baseline: 1.0136x; 1.0136x over previous
import functools

import jax
import jax.numpy as jnp
from jax import lax
from jax.experimental import pallas as pl
from jax.experimental.pallas import tpu as pltpu
from jax.experimental.pallas import tpu_sc as plsc

N = 10000
E = 320000
D = 128
R = 32
NB = 4

BLK = 400
NGRID = N // BLK

NWORKER = 32
CH = 112
NCHUNKS = (E + CH - 1) // CH
JTRIPS = (NCHUNKS + NWORKER - 1) // NWORKER
ZCH = 400
NZ = N // ZCH


def _y_body(h_ref, basis_ref, comp_ref, y_ref, w_scr):
    @pl.when(pl.program_id(0) == 0)
    def _():
        for r in range(R):
            w = comp_ref[r, 0] * basis_ref[0]
            for b in range(1, NB):
                w = w + comp_ref[r, b] * basis_ref[b]
            w_scr[r] = w.astype(jnp.bfloat16)

    h = h_ref[...].astype(jnp.bfloat16)
    for r in range(R):
        y_ref[:, r, :] = jnp.dot(h, w_scr[r], preferred_element_type=jnp.float32)


def _project(h, basis, comp):
    return pl.pallas_call(
        _y_body,
        grid=(NGRID,),
        in_specs=[
            pl.BlockSpec((BLK, D), lambda i: (i, 0)),
            pl.BlockSpec((NB, D, D), lambda i: (0, 0, 0)),
            pl.BlockSpec(memory_space=pltpu.SMEM),
        ],
        out_specs=pl.BlockSpec((BLK, R, D), lambda i: (i, 0, 0)),
        out_shape=jax.ShapeDtypeStruct((N, R, D), jnp.float32),
        scratch_shapes=[pltpu.VMEM((R, D, D), jnp.bfloat16)],
    )(h, basis, comp)


def _cy_body(h_ref, acc_ref, wself_ref, basis_ref, comp_ref, hn_ref, y_ref, w_scr):
    @pl.when(pl.program_id(0) == 0)
    def _():
        for r in range(R):
            w = comp_ref[r, 0] * basis_ref[0]
            for b in range(1, NB):
                w = w + comp_ref[r, b] * basis_ref[b]
            w_scr[r] = w.astype(jnp.bfloat16)

    hn = acc_ref[0] + acc_ref[1] + jnp.dot(
        h_ref[...], wself_ref[...], preferred_element_type=jnp.float32)
    hn = jnp.maximum(hn, 0.0)
    hn_ref[...] = hn
    hb = hn.astype(jnp.bfloat16)
    for r in range(R):
        y_ref[:, r, :] = jnp.dot(hb, w_scr[r], preferred_element_type=jnp.float32)


def _combine_project(h, acc, wself, basis, comp):
    return pl.pallas_call(
        _cy_body,
        grid=(NGRID,),
        in_specs=[
            pl.BlockSpec((BLK, D), lambda i: (i, 0)),
            pl.BlockSpec((2, BLK, D), lambda i: (0, i, 0)),
            pl.BlockSpec((D, D), lambda i: (0, 0)),
            pl.BlockSpec((NB, D, D), lambda i: (0, 0, 0)),
            pl.BlockSpec(memory_space=pltpu.SMEM),
        ],
        out_specs=[
            pl.BlockSpec((BLK, D), lambda i: (i, 0)),
            pl.BlockSpec((BLK, R, D), lambda i: (i, 0, 0)),
        ],
        out_shape=[
            jax.ShapeDtypeStruct((N, D), jnp.float32),
            jax.ShapeDtypeStruct((N, R, D), jnp.float32),
        ],
        scratch_shapes=[pltpu.VMEM((R, D, D), jnp.bfloat16)],
    )(h, acc, wself, basis, comp)


def _fin_body(h_ref, acc_ref, wself_ref, h3_ref, stats_ref, s1_scr, s2_scr):
    h3 = acc_ref[0] + acc_ref[1] + jnp.dot(
        h_ref[...], wself_ref[...], preferred_element_type=jnp.float32)
    h3 = jnp.maximum(h3, 0.0)
    h3_ref[...] = h3

    @pl.when(pl.program_id(0) == 0)
    def _():
        s1_scr[...] = jnp.zeros((8, D), jnp.float32)
        s2_scr[...] = jnp.zeros((8, D), jnp.float32)

    s1_scr[...] += jnp.sum(h3.reshape(BLK // 8, 8, D), axis=0)
    s2_scr[...] += jnp.sum((h3 * h3).reshape(BLK // 8, 8, D), axis=0)
    stats_ref[0] = s1_scr[...]
    stats_ref[1] = s2_scr[...]


def _finalize(h, acc, wself):
    return pl.pallas_call(
        _fin_body,
        grid=(NGRID,),
        in_specs=[
            pl.BlockSpec((BLK, D), lambda i: (i, 0)),
            pl.BlockSpec((2, BLK, D), lambda i: (0, i, 0)),
            pl.BlockSpec((D, D), lambda i: (0, 0)),
        ],
        out_specs=[
            pl.BlockSpec((BLK, D), lambda i: (i, 0)),
            pl.BlockSpec((2, 8, D), lambda i: (0, 0, 0)),
        ],
        out_shape=[
            jax.ShapeDtypeStruct((N, D), jnp.float32),
            jax.ShapeDtypeStruct((2, 8, D), jnp.float32),
        ],
        scratch_shapes=[
            pltpu.VMEM((8, D), jnp.float32),
            pltpu.VMEM((8, D), jnp.float32),
        ],
    )(h, acc, wself)


def _bn_body(h3_ref, h2_ref, stats_ref, gamma_ref, beta_ref, out_ref):
    inv_n = 1.0 / N
    mean = jnp.sum(stats_ref[0], axis=0, keepdims=True) * inv_n
    ex2 = jnp.sum(stats_ref[1], axis=0, keepdims=True) * inv_n
    var = ex2 - mean * mean
    inv = lax.rsqrt(var + 1e-5)
    out_ref[...] = h2_ref[...] + (h3_ref[...] - mean) * inv * gamma_ref[...] \
        + beta_ref[...]


def _batchnorm_residual(h3, h2, stats, gamma, beta):
    return pl.pallas_call(
        _bn_body,
        grid=(NGRID,),
        in_specs=[
            pl.BlockSpec((BLK, D), lambda i: (i, 0)),
            pl.BlockSpec((BLK, D), lambda i: (i, 0)),
            pl.BlockSpec((2, 8, D), lambda i: (0, 0, 0)),
            pl.BlockSpec((1, D), lambda i: (0, 0)),
            pl.BlockSpec((1, D), lambda i: (0, 0)),
        ],
        out_specs=pl.BlockSpec((BLK, D), lambda i: (i, 0)),
        out_shape=jax.ShapeDtypeStruct((N, D), jnp.float32),
    )(h3, h2, stats, gamma, beta)


_SC_MESH = plsc.VectorSubcoreMesh(core_axis_name="c", subcore_axis_name="s")
ROT = 3
_GSPLIT = ((0, CH),)


@functools.partial(
    pl.kernel,
    out_type=jax.ShapeDtypeStruct((2, N, D), jnp.float32),
    mesh=_SC_MESH,
    scratch_types=[
        pltpu.VMEM((4, CH), jnp.int32),
        pltpu.VMEM((4, CH), jnp.int32),
        pltpu.VMEM((4, CH), jnp.int32),
        pltpu.VMEM((CH,), jnp.int32),
        pltpu.VMEM((CH,), jnp.int32),
        pltpu.VMEM((CH,), jnp.int32),
        pltpu.VMEM((CH,), jnp.int32),
        pltpu.VMEM((CH,), jnp.int32),
        pltpu.VMEM((CH,), jnp.int32),
        pltpu.VMEM((CH, D), jnp.float32),
        pltpu.VMEM((CH, D), jnp.float32),
        pltpu.VMEM((CH, D), jnp.float32),
        pltpu.VMEM_SHARED((N, D), jnp.float32),
        pltpu.SemaphoreType.DMA,
        pltpu.SemaphoreType.DMA,
        pltpu.SemaphoreType.DMA,
        pltpu.SemaphoreType.DMA,
        pltpu.SemaphoreType.DMA,
        pltpu.SemaphoreType.DMA,
    ],
)
def _sc_edge_kernel(y_hbm, edata_hbm, zeros_hbm, out_hbm,
                    ed0, ed1, ed2, idx0, idx1, idx2, dst0, dst1, dst2,
                    rows0, rows1, rows2, acc_sh,
                    sg0, sg1, sg2, ss0, ss1, ss2):
    cid = lax.axis_index("c")
    sid = lax.axis_index("s")
    wid = cid * 16 + sid
    eds = (ed0, ed1, ed2)
    idxs = (idx0, idx1, idx2)
    dsts = (dst0, dst1, dst2)
    rows = (rows0, rows1, rows2)
    sgs = (sg0, sg1, sg2)
    sss = (ss0, ss1, ss2)

    for j in range((NZ + 15) // 16):
        zc = j * 16 + sid

        @pl.when(zc < NZ)
        def _(zc=zc):
            pltpu.sync_copy(zeros_hbm, acc_sh.at[pl.ds(zc * ZCH, ZCH)])

    plsc.subcore_barrier()

    def _prefetch(c, b):
        pltpu.sync_copy(edata_hbm.at[c], eds[b])

        def idx_body(i, cc):
            sl = pl.ds(i * 16, 16)
            idxs[b][sl] = eds[b][0, sl] * R + eds[b][1, sl]
            dsts[b][sl] = eds[b][2, sl]
            return cc

        lax.fori_loop(0, CH // 16, idx_body, 0, unroll=CH // 16)
        for (o, l) in _GSPLIT:
            pltpu.async_copy(y_hbm.at[idxs[b].at[pl.ds(o, l)]],
                             rows[b].at[pl.ds(o, l)], sgs[b])

    def _scale_and_scatter(b):
        for (o, l) in _GSPLIT:
            pltpu.make_async_copy(y_hbm.at[idxs[b].at[pl.ds(o, l)]],
                                  rows[b].at[pl.ds(o, l)], sgs[b]).wait()

        def scale_body(kk, cc):
            sl16 = pl.ds(kk * 16, 16)
            n16 = eds[b][3, sl16].astype(jnp.float32) * (1.0 / 16777216.0)
            for e in range(16):
                k = kk * 16 + e
                nv = n16[e]
                for j in range(D // 16):
                    sl = pl.ds(j * 16, 16)
                    rows[b][k, sl] = rows[b][k, sl] * nv
            return cc

        lax.fori_loop(0, CH // 16, scale_body, 0)
        pltpu.async_copy(rows[b], acc_sh.at[dsts[b]], sss[b], add=True)

    def _wait_scatter(b):
        pltpu.make_async_copy(rows[b], acc_sh.at[dsts[b]], sss[b]).wait()

    def chunk_body(j, carry):
        c = j * NWORKER + wid

        @pl.when(j == 0)
        def _():
            _prefetch(c, 0)
            _prefetch(c + NWORKER, 1)

        for b in range(ROT):
            is_b = lax.rem(j, ROT) == b
            bp = (b + 2) % ROT

            @pl.when(is_b & (j >= 1) & (c - NWORKER < NCHUNKS))
            def _(bp=bp):
                _wait_scatter(bp)

            @pl.when(is_b & (c + 2 * NWORKER < NCHUNKS))
            def _(c=c, bp=bp):
                _prefetch(c + 2 * NWORKER, bp)

            @pl.when(is_b & (c < NCHUNKS))
            def _(b=b):
                _scale_and_scatter(b)

        return carry

    lax.fori_loop(0, JTRIPS, chunk_body, 0)

    @pl.when((JTRIPS - 1) * NWORKER + wid < NCHUNKS)
    def _():
        _wait_scatter((JTRIPS - 1) % ROT)

    plsc.subcore_barrier()
    for j in range((NZ + 15) // 16):
        zc = j * 16 + sid

        @pl.when(zc < NZ)
        def _(zc=zc):
            pltpu.sync_copy(acc_sh.at[pl.ds(zc * ZCH, ZCH)],
                            out_hbm.at[cid, pl.ds(zc * ZCH, ZCH)])


def kernel(x, edge_index, edge_type, norm, basis0, comp0, wself0, basis1,
           comp1, wself1, basis2, comp2, wself2, gamma, beta):
    src = edge_index[0]
    dst = edge_index[1]
    zeros = jnp.zeros((ZCH, D), jnp.float32)
    normq = (norm * 16777216.0).astype(jnp.int32)
    pad = NCHUNKS * CH - E

    def _padded(a):
        return jnp.concatenate([a, jnp.zeros((pad,), jnp.int32)]).reshape(
            NCHUNKS, CH)

    edata = jnp.stack(
        [_padded(src), _padded(edge_type), _padded(dst), _padded(normq)],
        axis=1)

    y0 = _project(x, basis0, comp0)
    acc0 = _sc_edge_kernel(y0.reshape(N * R, D), edata, zeros)
    h1, y1 = _combine_project(x, acc0, wself0, basis1, comp1)
    acc1 = _sc_edge_kernel(y1.reshape(N * R, D), edata, zeros)
    h2, y2 = _combine_project(h1, acc1, wself1, basis2, comp2)
    acc2 = _sc_edge_kernel(y2.reshape(N * R, D), edata, zeros)
    h3, stats = _finalize(h2, acc2, wself2)
    out = _batchnorm_residual(h3, h2, stats, gamma.reshape(1, D),
                              beta.reshape(1, D))
    return out


# speedup vs baseline: 1.0138x; 1.0001x over previous
import functools

import jax
import jax.numpy as jnp
from jax import lax
from jax.experimental import pallas as pl
from jax.experimental.pallas import tpu as pltpu
from jax.experimental.pallas import tpu_sc as plsc

N = 10000
E = 320000
D = 128
R = 32
NB = 4

BLK = 400
NGRID = N // BLK

NWORKER = 32
CH = 112
NCHUNKS = (E + CH - 1) // CH
JTRIPS = (NCHUNKS + NWORKER - 1) // NWORKER
ZCH = 400
NZ = N // ZCH


def _y_body(h_ref, basis_ref, comp_ref, y_ref, w_scr):
    @pl.when(pl.program_id(0) == 0)
    def _():
        for r in range(R):
            w = comp_ref[r, 0] * basis_ref[0]
            for b in range(1, NB):
                w = w + comp_ref[r, b] * basis_ref[b]
            w_scr[r] = w.astype(jnp.bfloat16)

    h = h_ref[...].astype(jnp.bfloat16)
    for r in range(R):
        y_ref[:, r, :] = jnp.dot(h, w_scr[r], preferred_element_type=jnp.float32)


def _project(h, basis, comp):
    return pl.pallas_call(
        _y_body,
        grid=(NGRID,),
        in_specs=[
            pl.BlockSpec((BLK, D), lambda i: (i, 0)),
            pl.BlockSpec((NB, D, D), lambda i: (0, 0, 0)),
            pl.BlockSpec(memory_space=pltpu.SMEM),
        ],
        out_specs=pl.BlockSpec((BLK, R, D), lambda i: (i, 0, 0)),
        out_shape=jax.ShapeDtypeStruct((N, R, D), jnp.float32),
        scratch_shapes=[pltpu.VMEM((R, D, D), jnp.bfloat16)],
    )(h, basis, comp)


def _cy_body(h_ref, acc_ref, wself_ref, basis_ref, comp_ref, hn_ref, y_ref, w_scr):
    @pl.when(pl.program_id(0) == 0)
    def _():
        for r in range(R):
            w = comp_ref[r, 0] * basis_ref[0]
            for b in range(1, NB):
                w = w + comp_ref[r, b] * basis_ref[b]
            w_scr[r] = w.astype(jnp.bfloat16)

    hn = acc_ref[0] + acc_ref[1] + jnp.dot(
        h_ref[...], wself_ref[...], preferred_element_type=jnp.float32)
    hn = jnp.maximum(hn, 0.0)
    hn_ref[...] = hn
    hb = hn.astype(jnp.bfloat16)
    for r in range(R):
        y_ref[:, r, :] = jnp.dot(hb, w_scr[r], preferred_element_type=jnp.float32)


def _combine_project(h, acc, wself, basis, comp):
    return pl.pallas_call(
        _cy_body,
        grid=(NGRID,),
        in_specs=[
            pl.BlockSpec((BLK, D), lambda i: (i, 0)),
            pl.BlockSpec((2, BLK, D), lambda i: (0, i, 0)),
            pl.BlockSpec((D, D), lambda i: (0, 0)),
            pl.BlockSpec((NB, D, D), lambda i: (0, 0, 0)),
            pl.BlockSpec(memory_space=pltpu.SMEM),
        ],
        out_specs=[
            pl.BlockSpec((BLK, D), lambda i: (i, 0)),
            pl.BlockSpec((BLK, R, D), lambda i: (i, 0, 0)),
        ],
        out_shape=[
            jax.ShapeDtypeStruct((N, D), jnp.float32),
            jax.ShapeDtypeStruct((N, R, D), jnp.float32),
        ],
        scratch_shapes=[pltpu.VMEM((R, D, D), jnp.bfloat16)],
    )(h, acc, wself, basis, comp)


def _fin_body(h_ref, acc_ref, wself_ref, h3_ref, stats_ref, s1_scr, s2_scr):
    h3 = acc_ref[0] + acc_ref[1] + jnp.dot(
        h_ref[...], wself_ref[...], preferred_element_type=jnp.float32)
    h3 = jnp.maximum(h3, 0.0)
    h3_ref[...] = h3

    @pl.when(pl.program_id(0) == 0)
    def _():
        s1_scr[...] = jnp.zeros((8, D), jnp.float32)
        s2_scr[...] = jnp.zeros((8, D), jnp.float32)

    s1_scr[...] += jnp.sum(h3.reshape(BLK // 8, 8, D), axis=0)
    s2_scr[...] += jnp.sum((h3 * h3).reshape(BLK // 8, 8, D), axis=0)
    stats_ref[0] = s1_scr[...]
    stats_ref[1] = s2_scr[...]


def _finalize(h, acc, wself):
    return pl.pallas_call(
        _fin_body,
        grid=(NGRID,),
        in_specs=[
            pl.BlockSpec((BLK, D), lambda i: (i, 0)),
            pl.BlockSpec((2, BLK, D), lambda i: (0, i, 0)),
            pl.BlockSpec((D, D), lambda i: (0, 0)),
        ],
        out_specs=[
            pl.BlockSpec((BLK, D), lambda i: (i, 0)),
            pl.BlockSpec((2, 8, D), lambda i: (0, 0, 0)),
        ],
        out_shape=[
            jax.ShapeDtypeStruct((N, D), jnp.float32),
            jax.ShapeDtypeStruct((2, 8, D), jnp.float32),
        ],
        scratch_shapes=[
            pltpu.VMEM((8, D), jnp.float32),
            pltpu.VMEM((8, D), jnp.float32),
        ],
    )(h, acc, wself)


def _bn_body(h3_ref, h2_ref, stats_ref, gamma_ref, beta_ref, out_ref):
    inv_n = 1.0 / N
    mean = jnp.sum(stats_ref[0], axis=0, keepdims=True) * inv_n
    ex2 = jnp.sum(stats_ref[1], axis=0, keepdims=True) * inv_n
    var = ex2 - mean * mean
    inv = lax.rsqrt(var + 1e-5)
    out_ref[...] = h2_ref[...] + (h3_ref[...] - mean) * inv * gamma_ref[...] \
        + beta_ref[...]


def _batchnorm_residual(h3, h2, stats, gamma, beta):
    return pl.pallas_call(
        _bn_body,
        grid=(NGRID,),
        in_specs=[
            pl.BlockSpec((BLK, D), lambda i: (i, 0)),
            pl.BlockSpec((BLK, D), lambda i: (i, 0)),
            pl.BlockSpec((2, 8, D), lambda i: (0, 0, 0)),
            pl.BlockSpec((1, D), lambda i: (0, 0)),
            pl.BlockSpec((1, D), lambda i: (0, 0)),
        ],
        out_specs=pl.BlockSpec((BLK, D), lambda i: (i, 0)),
        out_shape=jax.ShapeDtypeStruct((N, D), jnp.float32),
    )(h3, h2, stats, gamma, beta)


_SC_MESH = plsc.VectorSubcoreMesh(core_axis_name="c", subcore_axis_name="s")
ROT = 3
_GSPLIT = ((0, CH),)


@functools.partial(
    pl.kernel,
    out_type=jax.ShapeDtypeStruct((2, N, D), jnp.float32),
    mesh=_SC_MESH,
    scratch_types=[
        pltpu.VMEM((4, CH), jnp.int32),
        pltpu.VMEM((4, CH), jnp.int32),
        pltpu.VMEM((4, CH), jnp.int32),
        pltpu.VMEM((CH,), jnp.int32),
        pltpu.VMEM((CH,), jnp.int32),
        pltpu.VMEM((CH,), jnp.int32),
        pltpu.VMEM((CH,), jnp.int32),
        pltpu.VMEM((CH,), jnp.int32),
        pltpu.VMEM((CH,), jnp.int32),
        pltpu.VMEM((CH, D), jnp.float32),
        pltpu.VMEM((CH, D), jnp.float32),
        pltpu.VMEM((CH, D), jnp.float32),
        pltpu.VMEM_SHARED((N, D), jnp.float32),
        pltpu.SemaphoreType.DMA,
        pltpu.SemaphoreType.DMA,
        pltpu.SemaphoreType.DMA,
        pltpu.SemaphoreType.DMA,
        pltpu.SemaphoreType.DMA,
        pltpu.SemaphoreType.DMA,
    ],
)
def _sc_edge_kernel(y_hbm, edata_hbm, zeros_hbm, out_hbm,
                    ed0, ed1, ed2, idx0, idx1, idx2, dst0, dst1, dst2,
                    rows0, rows1, rows2, acc_sh,
                    sg0, sg1, sg2, ss0, ss1, ss2):
    cid = lax.axis_index("c")
    sid = lax.axis_index("s")
    wid = cid * 16 + sid
    eds = (ed0, ed1, ed2)
    idxs = (idx0, idx1, idx2)
    dsts = (dst0, dst1, dst2)
    rows = (rows0, rows1, rows2)
    sgs = (sg0, sg1, sg2)
    sss = (ss0, ss1, ss2)

    def _prefetch(c, b):
        pltpu.sync_copy(edata_hbm.at[c], eds[b])

        def idx_body(i, cc):
            sl = pl.ds(i * 16, 16)
            idxs[b][sl] = eds[b][0, sl] * R + eds[b][1, sl]
            dsts[b][sl] = eds[b][2, sl]
            return cc

        lax.fori_loop(0, CH // 16, idx_body, 0, unroll=CH // 16)
        for (o, l) in _GSPLIT:
            pltpu.async_copy(y_hbm.at[idxs[b].at[pl.ds(o, l)]],
                             rows[b].at[pl.ds(o, l)], sgs[b])

    def _scale_and_scatter(b):
        for (o, l) in _GSPLIT:
            pltpu.make_async_copy(y_hbm.at[idxs[b].at[pl.ds(o, l)]],
                                  rows[b].at[pl.ds(o, l)], sgs[b]).wait()

        def scale_body(kk, cc):
            sl16 = pl.ds(kk * 16, 16)
            n16 = eds[b][3, sl16].astype(jnp.float32) * (1.0 / 16777216.0)
            for e in range(16):
                k = kk * 16 + e
                nv = n16[e]
                for j in range(D // 16):
                    sl = pl.ds(j * 16, 16)
                    rows[b][k, sl] = rows[b][k, sl] * nv
            return cc

        lax.fori_loop(0, CH // 16, scale_body, 0)
        pltpu.async_copy(rows[b], acc_sh.at[dsts[b]], sss[b], add=True)

    def _wait_scatter(b):
        pltpu.make_async_copy(rows[b], acc_sh.at[dsts[b]], sss[b]).wait()

    _prefetch(wid, 0)
    _prefetch(wid + NWORKER, 1)

    for j in range((NZ + 15) // 16):
        zc = j * 16 + sid

        @pl.when(zc < NZ)
        def _(zc=zc):
            pltpu.sync_copy(zeros_hbm, acc_sh.at[pl.ds(zc * ZCH, ZCH)])

    plsc.subcore_barrier()

    def chunk_body(j, carry):
        c = j * NWORKER + wid

        for b in range(ROT):
            is_b = lax.rem(j, ROT) == b
            bp = (b + 2) % ROT

            @pl.when(is_b & (j >= 1) & (c - NWORKER < NCHUNKS))
            def _(bp=bp):
                _wait_scatter(bp)

            @pl.when(is_b & (c + 2 * NWORKER < NCHUNKS))
            def _(c=c, bp=bp):
                _prefetch(c + 2 * NWORKER, bp)

            @pl.when(is_b & (c < NCHUNKS))
            def _(b=b):
                _scale_and_scatter(b)

        return carry

    lax.fori_loop(0, JTRIPS, chunk_body, 0)

    @pl.when((JTRIPS - 1) * NWORKER + wid < NCHUNKS)
    def _():
        _wait_scatter((JTRIPS - 1) % ROT)

    plsc.subcore_barrier()
    for j in range((NZ + 15) // 16):
        zc = j * 16 + sid

        @pl.when(zc < NZ)
        def _(zc=zc):
            pltpu.sync_copy(acc_sh.at[pl.ds(zc * ZCH, ZCH)],
                            out_hbm.at[cid, pl.ds(zc * ZCH, ZCH)])


def kernel(x, edge_index, edge_type, norm, basis0, comp0, wself0, basis1,
           comp1, wself1, basis2, comp2, wself2, gamma, beta):
    src = edge_index[0]
    dst = edge_index[1]
    zeros = jnp.zeros((ZCH, D), jnp.float32)
    normq = (norm * 16777216.0).astype(jnp.int32)
    pad = NCHUNKS * CH - E

    def _padded(a):
        return jnp.concatenate([a, jnp.zeros((pad,), jnp.int32)]).reshape(
            NCHUNKS, CH)

    edata = jnp.stack(
        [_padded(src), _padded(edge_type), _padded(dst), _padded(normq)],
        axis=1)

    y0 = _project(x, basis0, comp0)
    acc0 = _sc_edge_kernel(y0.reshape(N * R, D), edata, zeros)
    h1, y1 = _combine_project(x, acc0, wself0, basis1, comp1)
    acc1 = _sc_edge_kernel(y1.reshape(N * R, D), edata, zeros)
    h2, y2 = _combine_project(h1, acc1, wself1, basis2, comp2)
    acc2 = _sc_edge_kernel(y2.reshape(N * R, D), edata, zeros)
    h3, stats = _finalize(h2, acc2, wself2)
    out = _batchnorm_residual(h3, h2, stats, gamma.reshape(1, D),
                              beta.reshape(1, D))
    return out
